# baseline (speedup 1.0000x reference)
import functools

import jax
import jax.numpy as jnp
from jax import lax
from jax.experimental import pallas as pl
from jax.experimental.pallas import tpu as pltpu
from jax.experimental.pallas import tpu_sc as plsc

N = 10000
E = 320000
D = 128
_ROWS = 1000

_NS = 16
_EPT = E // _NS
_CAP = _EPT + 160
_DEGW = 20480
_WPT = _DEGW // _NS
_K = 128
_RRT = N // _NS


_SEG = 1280
_NSEG = 8
_PAD = 20000


def _sc_body(x_hbm, src_hbm, dst_hbm, ty_hbm, ew_hbm, agg_out, dinv_out,
             degp_out, src_v, fdst_v, ew_v, eidx_v, dg_v, acc_v, tmp_v, nrm_v,
             didx_v, didxb_v, gidx_v, rowbuf_v, agg_sh, semg, semh, sems):
    c = lax.axis_index("c")
    s = lax.axis_index("s")
    f32 = jnp.float32
    i32 = jnp.int32
    z16f = jnp.zeros((16,), f32)
    z16i = jnp.zeros((16,), i32)
    ones_m = jnp.ones((16,), jnp.bool_)
    iota16 = lax.iota(i32, 16)
    ebase = s * _EPT

    pltpu.sync_copy(src_hbm.at[pl.ds(ebase, _EPT)], src_v.at[pl.ds(0, _EPT)])
    pltpu.sync_copy(dst_hbm.at[pl.ds(ebase, _EPT)], fdst_v.at[pl.ds(0, _EPT)])
    pltpu.sync_copy(ty_hbm.at[pl.ds(ebase, _EPT)], eidx_v.at[pl.ds(0, _EPT)])
    pltpu.sync_copy(ew_hbm.at[pl.ds(ebase, _EPT)], ew_v.at[pl.ds(0, _EPT)])

    def ftr(i, _):
        sl = pl.ds(16 * i, 16)
        fdst_v[sl] = eidx_v[sl] * N + fdst_v[sl]
        return 0
    lax.fori_loop(0, _EPT // 16, ftr, 0, unroll=4)

    src_v[pl.ds(_PAD, 16)] = iota16 * 617 + s * 37
    ew_v[pl.ds(_PAD, 16)] = z16f

    def zdg(i, _):
        dg_v[pl.ds(16 * i, 16)] = z16f
        return 0
    lax.fori_loop(0, _DEGW // 16, zdg, 0, unroll=4)

    def dacc(i, _):
        sl = pl.ds(i * 16, 16)
        plsc.addupdate_scatter(dg_v, [fdst_v[sl]], ew_v[sl])
        return 0
    lax.fori_loop(0, _EPT // 16, dacc, 0, unroll=2)

    pltpu.sync_copy(dg_v, degp_out.at[pl.ds((c * _NS + s) * _DEGW, _DEGW)])
    plsc.subcore_barrier()

    cw = c * _NS * _DEGW
    r0 = s * _WPT
    pltpu.sync_copy(degp_out.at[pl.ds(cw + r0, _WPT)], acc_v)
    for t in range(1, _NS):
        pltpu.sync_copy(degp_out.at[pl.ds(cw + t * _DEGW + r0, _WPT)], tmp_v)

        def radd(i, _):
            sl = pl.ds(16 * i, 16)
            acc_v[sl] = acc_v[sl] + tmp_v[sl]
            return 0
        lax.fori_loop(0, _WPT // 16, radd, 0)

    def rsq(i, _):
        sl = pl.ds(16 * i, 16)
        v = acc_v[sl] + 1.0
        yi = 0x5F3759DF - lax.shift_right_logical(plsc.bitcast(v, i32), 1)
        y = plsc.bitcast(yi, f32)
        for _u in range(3):
            y = y * (1.5 - 0.5 * v * y * y)
        acc_v[sl] = y
        return 0
    lax.fori_loop(0, _WPT // 16, rsq, 0, unroll=2)

    pltpu.sync_copy(acc_v, degp_out.at[pl.ds(cw + r0, _WPT)])

    @pl.when(c == 0)
    def _():
        pltpu.sync_copy(acc_v, dinv_out.at[pl.ds(r0, _WPT)])

    plsc.subcore_barrier()
    pltpu.sync_copy(degp_out.at[pl.ds(cw, _DEGW)], dg_v)

    cN = c * N

    def seg(g, _):
        flo = cN + g * _SEG

        def zrb(k, _2):
            for j in range(8):
                rowbuf_v[k, pl.ds(16 * j, 16)] = z16f
            return 0
        lax.fori_loop(0, _K, zrb, 0, unroll=4)

        @pl.when(s < 10)
        def _():
            pltpu.sync_copy(rowbuf_v, agg_sh.at[pl.ds(s * 128, 128)])
        plsc.subcore_barrier()

        def comp(i, off):
            sl = pl.ds(i * 16, 16)
            fd = fdst_v[sl]
            msk = jnp.logical_and(fd >= flo, fd < flo + _SEG)
            plsc.store_compressed(eidx_v.at[pl.ds(off, 16)], iota16 + 16 * i,
                                  mask=msk)
            return off + jnp.sum(msk.astype(i32))
        n_g = lax.fori_loop(0, _EPT // 16, comp, jnp.int32(0), unroll=2)

        fdst_v[pl.ds(_PAD, 16)] = jnp.broadcast_to(flo, (16,))
        for j in range(9):
            plsc.store_compressed(eidx_v.at[pl.ds(n_g + 16 * j, 16)],
                                  iota16 + _PAD, mask=ones_m)

        nch = lax.div(n_g + (_K - 1), _K)

        def calc(ch, didx_r):
            cb = ch * _K
            for j in range(8):
                e16 = eidx_v[pl.ds(cb + 16 * j, 16)]
                s16 = plsc.load_gather(src_v, [e16])
                fd16 = plsc.load_gather(fdst_v, [e16])
                w16 = plsc.load_gather(ew_v, [e16])
                dvs = plsc.load_gather(dg_v, [cN + s16])
                dvd = plsc.load_gather(dg_v, [fd16])
                nrm_v[pl.ds(16 * j, 16)] = dvs * w16 * dvd
                didx_r[0, pl.ds(16 * j, 16)] = fd16 - flo
                gidx_v[0, pl.ds(16 * j, 16)] = s16

        @pl.when(nch > 0)
        def _():
            calc(0, didx_v)

        def chunk(ch, _2):
            p = lax.rem(ch, 2)
            for slot, didx_r in ((1, didx_v), (0, didxb_v)):
                @pl.when(jnp.logical_and(ch > 0, p == slot))
                def _(didx_r=didx_r):
                    pltpu.make_async_copy(rowbuf_v, agg_sh.at[didx_r.at[0]],
                                          sems).wait()
            pltpu.async_copy(x_hbm.at[gidx_v.at[0, pl.ds(0, 64)]],
                             rowbuf_v.at[pl.ds(0, 64)], semg)
            pltpu.async_copy(x_hbm.at[gidx_v.at[0, pl.ds(64, 64)]],
                             rowbuf_v.at[pl.ds(64, 64)], semh)

            def sk(i, _3):
                nv = nrm_v[pl.ds(16 * i, 16)]
                for l in range(16):
                    sc_ = nv[l]
                    row = 16 * i + l
                    for j2 in range(8):
                        sl2 = pl.ds(16 * j2, 16)
                        rowbuf_v[row, sl2] = rowbuf_v[row, sl2] * sc_
                return 0

            pltpu.make_async_copy(x_hbm.at[gidx_v.at[0, pl.ds(0, 64)]],
                                  rowbuf_v.at[pl.ds(0, 64)], semg).wait()
            lax.fori_loop(0, _K // 32, sk, 0)
            pltpu.make_async_copy(x_hbm.at[gidx_v.at[0, pl.ds(64, 64)]],
                                  rowbuf_v.at[pl.ds(64, 64)], semh).wait()
            lax.fori_loop(_K // 32, _K // 16, sk, 0)

            for slot, didx_r, didx_n in ((0, didx_v, didxb_v),
                                         (1, didxb_v, didx_v)):
                @pl.when(p == slot)
                def _(didx_r=didx_r, didx_n=didx_n):
                    pltpu.async_copy(rowbuf_v, agg_sh.at[didx_r.at[0]], sems,
                                     add=True)

                    @pl.when(ch + 1 < nch)
                    def _():
                        calc(ch + 1, didx_n)
            return 0

        lax.fori_loop(0, nch, chunk, 0)

        for slot, didx_r in ((0, didx_v), (1, didxb_v)):
            @pl.when(jnp.logical_and(nch > 0, lax.rem(nch - 1, 2) == slot))
            def _(didx_r=didx_r):
                pltpu.make_async_copy(rowbuf_v, agg_sh.at[didx_r.at[0]],
                                      sems).wait()
        plsc.subcore_barrier()

        gbase = g * _SEG

        @pl.when(jnp.logical_and(s < 10, gbase + s * 128 + 128 <= N))
        def _():
            pltpu.sync_copy(agg_sh.at[pl.ds(s * 128, 128)],
                            agg_out.at[c, pl.ds(gbase + s * 128, 128)])

        @pl.when(jnp.logical_and(s == 8, g == _NSEG - 1))
        def _():
            pltpu.sync_copy(agg_sh.at[pl.ds(1024, 16)],
                            agg_out.at[c, pl.ds(9984, 16)])
        plsc.subcore_barrier()
        return 0

    lax.fori_loop(0, _NSEG, seg, 0)


@jax.jit
def _sc_aggregate(x, src, dst, edge_type, edge_weight):
    f32 = jnp.float32
    mesh = plsc.VectorSubcoreMesh(core_axis_name="c", subcore_axis_name="s")
    return pl.kernel(
        _sc_body,
        out_type=[
            jax.ShapeDtypeStruct((2, N, D), f32),
            jax.ShapeDtypeStruct((_DEGW,), f32),
            jax.ShapeDtypeStruct((2 * _NS * _DEGW,), f32),
        ],
        mesh=mesh,
        compiler_params=pltpu.CompilerParams(needs_layout_passes=False),
        scratch_types=[
            pltpu.VMEM((_CAP,), jnp.int32),
            pltpu.VMEM((_CAP,), jnp.int32),
            pltpu.VMEM((_CAP,), f32),
            pltpu.VMEM((_CAP,), jnp.int32),
            pltpu.VMEM((_DEGW,), f32),
            pltpu.VMEM((_WPT,), f32),
            pltpu.VMEM((_WPT,), f32),
            pltpu.VMEM((_K,), f32),
            pltpu.VMEM((1, _K), jnp.int32),
            pltpu.VMEM((1, _K), jnp.int32),
            pltpu.VMEM((1, _K), jnp.int32),
            pltpu.VMEM((_K, D), f32),
            pltpu.VMEM_SHARED((_SEG, D), f32),
            pltpu.SemaphoreType.DMA,
            pltpu.SemaphoreType.DMA,
            pltpu.SemaphoreType.DMA,
        ],
    )(x, src, dst, edge_type, edge_weight)


def _dense_body(x_ref, a0_ref, a1_ref, dv0_ref, dv1_ref, wsl_ref, w0_ref,
                w1_ref, wg_ref, bsl_ref, b0_ref, b1_ref, bg_ref, out_ref):
    f32 = jnp.float32
    dot = functools.partial(jax.lax.dot_general,
                            dimension_numbers=(((1,), (0,)), ((), ())),
                            precision=jax.lax.Precision.HIGHEST,
                            preferred_element_type=f32)
    xb = x_ref[...]
    dv0 = dv0_ref[...]
    dv1 = dv1_ref[...]
    a0 = a0_ref[...] + xb * (dv0 * dv0)
    a1 = a1_ref[...] + xb * (dv1 * dv1)
    xx = dot(xb, wsl_ref[...]) + bsl_ref[...]
    h0 = dot(a0, w0_ref[...]) + b0_ref[...]
    h1 = dot(a1, w1_ref[...]) + b1_ref[...]
    g = (dot(xx, wg_ref[0:D]) + dot(h0, wg_ref[D:2 * D])
         + dot(h1, wg_ref[2 * D:3 * D]) + bg_ref[...])
    m = jnp.max(g, axis=-1, keepdims=True)
    p = jnp.exp(g - m)
    sft = p / jnp.sum(p, axis=-1, keepdims=True)
    r = jax.lax.broadcasted_iota(jnp.int32, (D, D), 0)
    col = jax.lax.broadcasted_iota(jnp.int32, (D, D), 1)
    tri = (r <= col).astype(f32)
    exc = (r + col == D - 1).astype(f32)
    gat = dot(sft, tri)
    rev = dot(h1, exc)
    out_ref[...] = rev * gat + xx + h0


def _dense_stage(x, agg0, agg1, dinv0, dinv1, W_sl, W0, W1, Wg, b_sl, b0, b1,
                 bg):
    grid = (N // _ROWS,)
    row_spec = pl.BlockSpec((_ROWS, D), lambda i: (i, 0))
    col_spec = pl.BlockSpec((_ROWS, 1), lambda i: (i, 0))
    full = lambda shape: pl.BlockSpec(shape, lambda i: (0, 0))
    return pl.pallas_call(
        _dense_body,
        grid=grid,
        in_specs=[
            row_spec, row_spec, row_spec, col_spec, col_spec,
            full((D, D)), full((D, D)), full((D, D)), full((3 * D, D)),
            full((1, D)), full((1, D)), full((1, D)), full((1, D)),
        ],
        out_specs=row_spec,
        out_shape=jax.ShapeDtypeStruct((N, D), jnp.float32),
    )(x, agg0, agg1, dinv0, dinv1, W_sl, W0, W1, Wg,
      b_sl.reshape(1, D), b0.reshape(1, D), b1.reshape(1, D),
      bg.reshape(1, D))


def kernel(x, edge_index, edge_type, edge_weight, W_sl, b_sl, W0, b0, W1, b1,
           Wg, bg):
    agg, dinv_rows, _degp = _sc_aggregate(x, edge_index[0], edge_index[1],
                                          edge_type, edge_weight)
    dinv = dinv_rows.reshape(-1)[:2 * N].reshape(2, N)
    return _dense_stage(x, agg[0], agg[1], dinv[0].reshape(N, 1),
                        dinv[1].reshape(N, 1), W_sl, W0, W1, Wg, b_sl, b0, b1,
                        bg)


# speedup vs baseline: 1.0508x; 1.0508x over previous
import functools

import jax
import jax.numpy as jnp
from jax import lax
from jax.experimental import pallas as pl
from jax.experimental.pallas import tpu as pltpu
from jax.experimental.pallas import tpu_sc as plsc

N = 10000
E = 320000
D = 128
_ROWS = 1000

_NS = 16
_EPT = E // _NS
_CAP = _EPT + 160
_DEGW = 20480
_WPT = _DEGW // _NS
_K = 128
_RRT = N // _NS


_SEG = 1280
_NSEG = 8
_PAD = 20000


def _sc_body(x_hbm, src_hbm, dst_hbm, ty_hbm, ew_hbm, agg_out, dinv_out,
             degp_out, src_v, fdst_v, ew_v, eidx_v, dg_v, acc_v, tmp_v, nrm_v,
             didx_v, didxb_v, gidx_v, rowbuf_v, agg_sh, semg, semh, sems):
    c = lax.axis_index("c")
    s = lax.axis_index("s")
    f32 = jnp.float32
    i32 = jnp.int32
    z16f = jnp.zeros((16,), f32)
    z16i = jnp.zeros((16,), i32)
    ones_m = jnp.ones((16,), jnp.bool_)
    iota16 = lax.iota(i32, 16)
    ebase = s * _EPT

    pltpu.sync_copy(src_hbm.at[pl.ds(ebase, _EPT)], src_v.at[pl.ds(0, _EPT)])
    pltpu.sync_copy(dst_hbm.at[pl.ds(ebase, _EPT)], fdst_v.at[pl.ds(0, _EPT)])
    pltpu.sync_copy(ty_hbm.at[pl.ds(ebase, _EPT)], eidx_v.at[pl.ds(0, _EPT)])
    pltpu.sync_copy(ew_hbm.at[pl.ds(ebase, _EPT)], ew_v.at[pl.ds(0, _EPT)])

    def ftr(i, _):
        sl = pl.ds(16 * i, 16)
        fdst_v[sl] = eidx_v[sl] * N + fdst_v[sl]
        return 0
    lax.fori_loop(0, _EPT // 16, ftr, 0, unroll=4)

    src_v[pl.ds(_PAD, 16)] = iota16 * 617 + s * 37
    ew_v[pl.ds(_PAD, 16)] = z16f

    def zdg(i, _):
        dg_v[pl.ds(16 * i, 16)] = z16f
        return 0
    lax.fori_loop(0, _DEGW // 16, zdg, 0, unroll=4)

    def dacc(i, _):
        sl = pl.ds(i * 16, 16)
        plsc.addupdate_scatter(dg_v, [fdst_v[sl]], ew_v[sl])
        return 0
    lax.fori_loop(0, _EPT // 16, dacc, 0)

    pltpu.sync_copy(dg_v, degp_out.at[pl.ds((c * _NS + s) * _DEGW, _DEGW)])
    plsc.subcore_barrier()

    cw = c * _NS * _DEGW
    r0 = s * _WPT
    pltpu.sync_copy(degp_out.at[pl.ds(cw + r0, _WPT)], acc_v)
    for t in range(1, _NS):
        pltpu.sync_copy(degp_out.at[pl.ds(cw + t * _DEGW + r0, _WPT)], tmp_v)

        def radd(i, _):
            sl = pl.ds(16 * i, 16)
            acc_v[sl] = acc_v[sl] + tmp_v[sl]
            return 0
        lax.fori_loop(0, _WPT // 16, radd, 0)

    def rsq(i, _):
        sl = pl.ds(16 * i, 16)
        v = acc_v[sl] + 1.0
        yi = 0x5F3759DF - lax.shift_right_logical(plsc.bitcast(v, i32), 1)
        y = plsc.bitcast(yi, f32)
        for _u in range(3):
            y = y * (1.5 - 0.5 * v * y * y)
        acc_v[sl] = y
        return 0
    lax.fori_loop(0, _WPT // 16, rsq, 0, unroll=2)

    pltpu.sync_copy(acc_v, degp_out.at[pl.ds(cw + r0, _WPT)])

    @pl.when(c == 0)
    def _():
        pltpu.sync_copy(acc_v, dinv_out.at[pl.ds(r0, _WPT)])

    plsc.subcore_barrier()
    pltpu.sync_copy(degp_out.at[pl.ds(cw, _DEGW)], dg_v)

    cN = c * N

    def seg(g, _):
        flo = cN + g * _SEG

        def zrb(k, _2):
            for j in range(8):
                rowbuf_v[k, pl.ds(16 * j, 16)] = z16f
            return 0
        lax.fori_loop(0, _K, zrb, 0, unroll=4)

        @pl.when(s < 10)
        def _():
            pltpu.sync_copy(rowbuf_v, agg_sh.at[pl.ds(s * 128, 128)])
        plsc.subcore_barrier()

        def comp(i, off):
            sl = pl.ds(i * 16, 16)
            fd = fdst_v[sl]
            msk = jnp.logical_and(fd >= flo, fd < flo + _SEG)
            plsc.store_compressed(eidx_v.at[pl.ds(off, 16)], iota16 + 16 * i,
                                  mask=msk)
            return off + jnp.sum(msk.astype(i32))
        n_g = lax.fori_loop(0, _EPT // 16, comp, jnp.int32(0))

        fdst_v[pl.ds(_PAD, 16)] = jnp.broadcast_to(flo, (16,))
        for j in range(9):
            plsc.store_compressed(eidx_v.at[pl.ds(n_g + 16 * j, 16)],
                                  iota16 + _PAD, mask=ones_m)

        nch = lax.div(n_g + (_K - 1), _K)

        def calc(ch, didx_r):
            cb = ch * _K
            for j in range(8):
                e16 = eidx_v[pl.ds(cb + 16 * j, 16)]
                s16 = plsc.load_gather(src_v, [e16])
                fd16 = plsc.load_gather(fdst_v, [e16])
                w16 = plsc.load_gather(ew_v, [e16])
                dvs = plsc.load_gather(dg_v, [cN + s16])
                dvd = plsc.load_gather(dg_v, [fd16])
                nrm_v[pl.ds(16 * j, 16)] = dvs * w16 * dvd
                didx_r[0, pl.ds(16 * j, 16)] = fd16 - flo
                gidx_v[0, pl.ds(16 * j, 16)] = s16

        @pl.when(nch > 0)
        def _():
            calc(0, didx_v)

        def chunk(ch, _2):
            p = lax.rem(ch, 2)
            for slot, didx_r in ((1, didx_v), (0, didxb_v)):
                @pl.when(jnp.logical_and(ch > 0, p == slot))
                def _(didx_r=didx_r):
                    pltpu.make_async_copy(rowbuf_v, agg_sh.at[didx_r.at[0]],
                                          sems).wait()
            pltpu.async_copy(x_hbm.at[gidx_v.at[0, pl.ds(0, 64)]],
                             rowbuf_v.at[pl.ds(0, 64)], semg)
            pltpu.async_copy(x_hbm.at[gidx_v.at[0, pl.ds(64, 64)]],
                             rowbuf_v.at[pl.ds(64, 64)], semh)

            def sk(i, _3):
                nv = nrm_v[pl.ds(16 * i, 16)]
                for l in range(16):
                    sc_ = nv[l]
                    row = 16 * i + l
                    for j2 in range(8):
                        sl2 = pl.ds(16 * j2, 16)
                        rowbuf_v[row, sl2] = rowbuf_v[row, sl2] * sc_
                return 0

            pltpu.make_async_copy(x_hbm.at[gidx_v.at[0, pl.ds(0, 64)]],
                                  rowbuf_v.at[pl.ds(0, 64)], semg).wait()
            lax.fori_loop(0, _K // 32, sk, 0)
            pltpu.make_async_copy(x_hbm.at[gidx_v.at[0, pl.ds(64, 64)]],
                                  rowbuf_v.at[pl.ds(64, 64)], semh).wait()
            lax.fori_loop(_K // 32, _K // 16, sk, 0)

            for slot, didx_r, didx_n in ((0, didx_v, didxb_v),
                                         (1, didxb_v, didx_v)):
                @pl.when(p == slot)
                def _(didx_r=didx_r, didx_n=didx_n):
                    pltpu.async_copy(rowbuf_v, agg_sh.at[didx_r.at[0]], sems,
                                     add=True)

                    @pl.when(ch + 1 < nch)
                    def _():
                        calc(ch + 1, didx_n)
            return 0

        lax.fori_loop(0, nch, chunk, 0)

        for slot, didx_r in ((0, didx_v), (1, didxb_v)):
            @pl.when(jnp.logical_and(nch > 0, lax.rem(nch - 1, 2) == slot))
            def _(didx_r=didx_r):
                pltpu.make_async_copy(rowbuf_v, agg_sh.at[didx_r.at[0]],
                                      sems).wait()
        plsc.subcore_barrier()

        gbase = g * _SEG

        @pl.when(jnp.logical_and(s < 10, gbase + s * 128 + 128 <= N))
        def _():
            pltpu.sync_copy(agg_sh.at[pl.ds(s * 128, 128)],
                            agg_out.at[c, pl.ds(gbase + s * 128, 128)])

        @pl.when(jnp.logical_and(s == 8, g == _NSEG - 1))
        def _():
            pltpu.sync_copy(agg_sh.at[pl.ds(1024, 16)],
                            agg_out.at[c, pl.ds(9984, 16)])
        plsc.subcore_barrier()
        return 0

    lax.fori_loop(0, _NSEG, seg, 0)


@jax.jit
def _sc_aggregate(x, src, dst, edge_type, edge_weight):
    f32 = jnp.float32
    mesh = plsc.VectorSubcoreMesh(core_axis_name="c", subcore_axis_name="s")
    return pl.kernel(
        _sc_body,
        out_type=[
            jax.ShapeDtypeStruct((2, N, D), f32),
            jax.ShapeDtypeStruct((_DEGW,), f32),
            jax.ShapeDtypeStruct((2 * _NS * _DEGW,), f32),
        ],
        mesh=mesh,
        compiler_params=pltpu.CompilerParams(needs_layout_passes=False),
        scratch_types=[
            pltpu.VMEM((_CAP,), jnp.int32),
            pltpu.VMEM((_CAP,), jnp.int32),
            pltpu.VMEM((_CAP,), f32),
            pltpu.VMEM((_CAP,), jnp.int32),
            pltpu.VMEM((_DEGW,), f32),
            pltpu.VMEM((_WPT,), f32),
            pltpu.VMEM((_WPT,), f32),
            pltpu.VMEM((_K,), f32),
            pltpu.VMEM((1, _K), jnp.int32),
            pltpu.VMEM((1, _K), jnp.int32),
            pltpu.VMEM((1, _K), jnp.int32),
            pltpu.VMEM((_K, D), f32),
            pltpu.VMEM_SHARED((_SEG, D), f32),
            pltpu.SemaphoreType.DMA,
            pltpu.SemaphoreType.DMA,
            pltpu.SemaphoreType.DMA,
        ],
    )(x, src, dst, edge_type, edge_weight)


def _dense_body(x_ref, a0_ref, a1_ref, dv0_ref, dv1_ref, wsl_ref, w0_ref,
                w1_ref, wg_ref, bsl_ref, b0_ref, b1_ref, bg_ref, out_ref):
    f32 = jnp.float32
    dot = functools.partial(jax.lax.dot_general,
                            dimension_numbers=(((1,), (0,)), ((), ())),
                            precision=jax.lax.Precision.HIGHEST,
                            preferred_element_type=f32)
    xb = x_ref[...]
    dv0 = dv0_ref[...]
    dv1 = dv1_ref[...]
    a0 = a0_ref[...] + xb * (dv0 * dv0)
    a1 = a1_ref[...] + xb * (dv1 * dv1)
    xx = dot(xb, wsl_ref[...]) + bsl_ref[...]
    h0 = dot(a0, w0_ref[...]) + b0_ref[...]
    h1 = dot(a1, w1_ref[...]) + b1_ref[...]
    g = (dot(xx, wg_ref[0:D]) + dot(h0, wg_ref[D:2 * D])
         + dot(h1, wg_ref[2 * D:3 * D]) + bg_ref[...])
    m = jnp.max(g, axis=-1, keepdims=True)
    p = jnp.exp(g - m)
    sft = p / jnp.sum(p, axis=-1, keepdims=True)
    r = jax.lax.broadcasted_iota(jnp.int32, (D, D), 0)
    col = jax.lax.broadcasted_iota(jnp.int32, (D, D), 1)
    tri = (r <= col).astype(f32)
    exc = (r + col == D - 1).astype(f32)
    gat = dot(sft, tri)
    rev = dot(h1, exc)
    out_ref[...] = rev * gat + xx + h0


def _dense_stage(x, agg0, agg1, dinv0, dinv1, W_sl, W0, W1, Wg, b_sl, b0, b1,
                 bg):
    grid = (N // _ROWS,)
    row_spec = pl.BlockSpec((_ROWS, D), lambda i: (i, 0))
    col_spec = pl.BlockSpec((_ROWS, 1), lambda i: (i, 0))
    full = lambda shape: pl.BlockSpec(shape, lambda i: (0, 0))
    return pl.pallas_call(
        _dense_body,
        grid=grid,
        in_specs=[
            row_spec, row_spec, row_spec, col_spec, col_spec,
            full((D, D)), full((D, D)), full((D, D)), full((3 * D, D)),
            full((1, D)), full((1, D)), full((1, D)), full((1, D)),
        ],
        out_specs=row_spec,
        out_shape=jax.ShapeDtypeStruct((N, D), jnp.float32),
    )(x, agg0, agg1, dinv0, dinv1, W_sl, W0, W1, Wg,
      b_sl.reshape(1, D), b0.reshape(1, D), b1.reshape(1, D),
      bg.reshape(1, D))


def kernel(x, edge_index, edge_type, edge_weight, W_sl, b_sl, W0, b0, W1, b1,
           Wg, bg):
    agg, dinv_rows, _degp = _sc_aggregate(x, edge_index[0], edge_index[1],
                                          edge_type, edge_weight)
    dinv = dinv_rows.reshape(-1)[:2 * N].reshape(2, N)
    return _dense_stage(x, agg[0], agg[1], dinv[0].reshape(N, 1),
                        dinv[1].reshape(N, 1), W_sl, W0, W1, Wg, b_sl, b0, b1,
                        bg)
